# Initial kernel scaffold; baseline (speedup 1.0000x reference)
import functools

import jax
import jax.numpy as jnp
from jax import lax
from jax.experimental import pallas as pl
from jax.experimental.pallas import tpu as pltpu
from jax.experimental.pallas import tpu_sc as plsc

N = 10000
E = 320000
D = 128
DE = 16
EPS = 1e-5

NC = 2
NS = 16
NW = NC * NS
EPW = E // NW
K = 80
NCHUNK = EPW // K
ROWS_PT = N // NS

_HIGH = lax.Precision.HIGHEST


def _prep_body(x_ref, awt, ab, bwt, bb, dwt, db, ewt, eb_, ax_out, dtab_out,
               ebtab_out):
    x = x_ref[...]
    ax_out[...] = jnp.dot(x, awt[...], precision=_HIGH) + ab[...]
    dtab_out[...] = jnp.dot(x, dwt[...], precision=_HIGH) + db[...]
    ebtab_out[:, :D] = jnp.dot(x, ewt[...], precision=_HIGH) + eb_[...]
    ebtab_out[:, D:] = jnp.dot(x, bwt[...], precision=_HIGH) + bb[...]


def _ce_body(attr_ref, cwt_ref, cb_ref, ce_out):
    ce_out[...] = (
        jnp.dot(attr_ref[...], cwt_ref[...], precision=_HIGH) + cb_ref[...]
    )


def _sc_edge_body(row_hbm, col_hbm, dtab_hbm, ebtab_hbm, ce_hbm,
                  eij_hbm, aggr_hbm, stats_hbm,
                  row_v, col_v, gd_v, geb_v, ce_v, msg_v, st_v, aggr_sp,
                  sem0, sem1, sem2):
    c = lax.axis_index("c")
    s = lax.axis_index("s")
    wid = c * NS + s

    def _zero_row(r, _):
        for j in range(D // 16):
            msg_v[r, pl.ds(j * 16, 16)] = jnp.zeros((16,), jnp.float32)
        return 0

    lax.fori_loop(0, K, _zero_row, 0)
    base_row = s * ROWS_PT
    nfull = ROWS_PT // K
    rem = ROWS_PT - nfull * K
    for t in range(nfull):
        pltpu.sync_copy(msg_v, aggr_sp.at[pl.ds(base_row + t * K, K)])
    pltpu.sync_copy(msg_v.at[pl.ds(0, rem)],
                    aggr_sp.at[pl.ds(base_row + nfull * K, rem)])
    plsc.subcore_barrier()

    e0 = wid * EPW

    def _chunk(ci, carry):
        base = e0 + ci * K
        pltpu.sync_copy(row_hbm.at[pl.ds(base, K)], row_v)
        pltpu.sync_copy(col_hbm.at[pl.ds(base, K)], col_v)
        cp0 = pltpu.async_copy(dtab_hbm.at[row_v], gd_v, sem0)
        cp1 = pltpu.async_copy(ebtab_hbm.at[col_v], geb_v, sem1)
        cp2 = pltpu.async_copy(ce_hbm.at[pl.ds(base, K)], ce_v, sem2)
        cp0.wait()
        cp1.wait()
        cp2.wait()

        def _row(r, rc):
            vs = list(rc)
            for j in range(D // 16):
                sl = pl.ds(j * 16, 16)
                e = gd_v[r, sl] + geb_v[r, sl] + ce_v[r, sl]
                gd_v[r, sl] = e
                sig = 1.0 / (1.0 + jnp.exp(-e))
                msg_v[r, sl] = sig * geb_v[r, pl.ds(D + j * 16, 16)]
                vs[j] = vs[j] + e
                vs[8 + j] = vs[8 + j] + e * e
            return tuple(vs)

        carry = lax.fori_loop(0, K, _row, carry)
        pltpu.sync_copy(gd_v, eij_hbm.at[pl.ds(base, K)])
        pltpu.sync_copy(msg_v, aggr_sp.at[row_v], add=True)
        return carry

    zeros = tuple(jnp.zeros((16,), jnp.float32) for _ in range(16))
    sums = lax.fori_loop(0, NCHUNK, _chunk, zeros)

    for j in range(D // 16):
        st_v[0, pl.ds(j * 16, 16)] = sums[j]
        st_v[1, pl.ds(j * 16, 16)] = sums[8 + j]
    pltpu.sync_copy(st_v, stats_hbm.at[wid])

    plsc.subcore_barrier()
    pltpu.sync_copy(aggr_sp.at[pl.ds(base_row, ROWS_PT)],
                    aggr_hbm.at[c, pl.ds(base_row, ROWS_PT)])


def _node_body(x_ref, ax_ref, agg_ref, g_ref, b_ref, out_ref):
    t = ax_ref[...] + agg_ref[0] + agg_ref[1]
    m = jnp.mean(t, axis=0, keepdims=True)
    d = t - m
    v = jnp.mean(d * d, axis=0, keepdims=True)
    y = d * lax.rsqrt(v + EPS) * g_ref[...] + b_ref[...]
    out_ref[...] = x_ref[...] + jnp.maximum(y, 0.0)


def _efin_body(eij_ref, attr_ref, rwt_ref, stats_ref, g_ref, b_ref, out_ref):
    ssum = jnp.sum(stats_ref[:, 0, :], axis=0, keepdims=True)
    ssq = jnp.sum(stats_ref[:, 1, :], axis=0, keepdims=True)
    m = ssum * (1.0 / E)
    v = ssq * (1.0 / E) - m * m
    scale = lax.rsqrt(v + EPS) * g_ref[...]
    y = (eij_ref[...] - m) * scale + b_ref[...]
    out_ref[...] = (
        jnp.dot(attr_ref[...], rwt_ref[...], precision=_HIGH)
        + jnp.maximum(y, 0.0)
    )


def kernel(x_in_node, edge_idx, edge_in_attr, A_w, A_b, B_w, B_b, C_w, C_b,
           D_w, D_b, E_w, E_b, bn_x_g, bn_x_b, bn_e_g, bn_e_b, res_e_w):
    row = jnp.asarray(edge_idx[0], jnp.int32)
    col = jnp.asarray(edge_idx[1], jnp.int32)

    ax, dtab, ebtab = pl.pallas_call(
        _prep_body,
        out_shape=(
            jax.ShapeDtypeStruct((N, D), jnp.float32),
            jax.ShapeDtypeStruct((N, D), jnp.float32),
            jax.ShapeDtypeStruct((N, 2 * D), jnp.float32),
        ),
    )(x_in_node, A_w.T, A_b[None, :], B_w.T, B_b[None, :], D_w.T,
      D_b[None, :], E_w.T, E_b[None, :])

    BE = 16000
    ce = pl.pallas_call(
        _ce_body,
        grid=(E // BE,),
        in_specs=[
            pl.BlockSpec((BE, DE), lambda i: (i, 0)),
            pl.BlockSpec((DE, D), lambda i: (0, 0)),
            pl.BlockSpec((1, D), lambda i: (0, 0)),
        ],
        out_specs=pl.BlockSpec((BE, D), lambda i: (i, 0)),
        out_shape=jax.ShapeDtypeStruct((E, D), jnp.float32),
    )(edge_in_attr, C_w.T, C_b[None, :])

    sc_edge = functools.partial(
        pl.kernel,
        out_type=(
            jax.ShapeDtypeStruct((E, D), jnp.float32),
            jax.ShapeDtypeStruct((NC, N, D), jnp.float32),
            jax.ShapeDtypeStruct((NW, 2, D), jnp.float32),
        ),
        mesh=plsc.VectorSubcoreMesh(core_axis_name="c", subcore_axis_name="s"),
        scratch_types=[
            pltpu.VMEM((K,), jnp.int32),
            pltpu.VMEM((K,), jnp.int32),
            pltpu.VMEM((K, D), jnp.float32),
            pltpu.VMEM((K, 2 * D), jnp.float32),
            pltpu.VMEM((K, D), jnp.float32),
            pltpu.VMEM((K, D), jnp.float32),
            pltpu.VMEM((2, D), jnp.float32),
            pltpu.VMEM_SHARED((N, D), jnp.float32),
            pltpu.SemaphoreType.DMA,
            pltpu.SemaphoreType.DMA,
            pltpu.SemaphoreType.DMA,
        ],
    )(_sc_edge_body)
    eij, aggr, stats = sc_edge(row, col, dtab, ebtab, ce)

    x_final = pl.pallas_call(
        _node_body,
        out_shape=jax.ShapeDtypeStruct((N, D), jnp.float32),
    )(x_in_node, ax, aggr, bn_x_g[None, :], bn_x_b[None, :])

    BF = 8000
    e_final = pl.pallas_call(
        _efin_body,
        grid=(E // BF,),
        in_specs=[
            pl.BlockSpec((BF, D), lambda i: (i, 0)),
            pl.BlockSpec((BF, DE), lambda i: (i, 0)),
            pl.BlockSpec((DE, D), lambda i: (0, 0)),
            pl.BlockSpec((NW, 2, D), lambda i: (0, 0, 0)),
            pl.BlockSpec((1, D), lambda i: (0, 0)),
            pl.BlockSpec((1, D), lambda i: (0, 0)),
        ],
        out_specs=pl.BlockSpec((BF, D), lambda i: (i, 0)),
        out_shape=jax.ShapeDtypeStruct((E, D), jnp.float32),
    )(eij, edge_in_attr, res_e_w.T, stats, bn_e_g[None, :], bn_e_b[None, :])

    return (x_final, e_final)


# speedup vs baseline: 1.2271x; 1.2271x over previous
import functools

import jax
import jax.numpy as jnp
from jax import lax
from jax.experimental import pallas as pl
from jax.experimental.pallas import tpu as pltpu
from jax.experimental.pallas import tpu_sc as plsc

N = 10000
E = 320000
D = 128
DE = 16
EPS = 1e-5

NC = 2
NS = 16
NW = NC * NS
EPW = E // NW
K = 80
NCHUNK = EPW // K
ROWS_PT = 640

_HIGH = lax.Precision.HIGHEST


def _prep_body(x_ref, awt, ab, bwt, bb, dwt, db, ewt, eb_, ax_out, dtab_out,
               ebtab_out):
    x = x_ref[...]
    ax_out[...] = jnp.dot(x, awt[...], precision=_HIGH) + ab[...]
    dtab_out[...] = jnp.dot(x, dwt[...], precision=_HIGH) + db[...]
    ebtab_out[:, :D] = jnp.dot(x, ewt[...], precision=_HIGH) + eb_[...]
    ebtab_out[:, D:] = jnp.dot(x, bwt[...], precision=_HIGH) + bb[...]


def _ce_body(attr_ref, cwt_ref, cb_ref, ce_out):
    ce_out[...] = (
        jnp.dot(attr_ref[...], cwt_ref[...], precision=_HIGH) + cb_ref[...]
    )


def _sc_edge_body(row_hbm, col_hbm, dtab_hbm, ebtab_hbm, ce_hbm,
                  eij_hbm, aggr_hbm, stats_hbm,
                  row_v, col_v, gd_v, geb_v, msg_v, st_v, aggr_sp,
                  sem0, sem1, sem2):
    c = lax.axis_index("c")
    s = lax.axis_index("s")
    wid = c * NS + s

    def _zero_row(r, _):
        for j in range(D // 16):
            msg_v[r, pl.ds(j * 16, 16)] = jnp.zeros((16,), jnp.float32)
        return 0

    lax.fori_loop(0, K, _zero_row, 0)
    base_row = s * ROWS_PT
    nblk = jnp.where(s == NS - 1, (N - (NS - 1) * ROWS_PT) // K,
                     ROWS_PT // K)

    def _zero_blk(t, _):
        pltpu.sync_copy(msg_v, aggr_sp.at[pl.ds(base_row + t * K, K)])
        return 0

    lax.fori_loop(0, nblk, _zero_blk, 0)
    plsc.subcore_barrier()

    e0 = wid * EPW

    def _chunk(ci, carry):
        base = e0 + ci * K
        pltpu.sync_copy(row_hbm.at[pl.ds(base, K)], row_v)
        pltpu.sync_copy(col_hbm.at[pl.ds(base, K)], col_v)
        cp0 = pltpu.async_copy(dtab_hbm.at[row_v], gd_v, sem0)
        cp1 = pltpu.async_copy(ebtab_hbm.at[col_v], geb_v, sem1)
        cp2 = pltpu.async_copy(ce_hbm.at[pl.ds(base, K)], msg_v, sem2)
        cp0.wait()
        cp1.wait()
        cp2.wait()

        def _row(r, rc):
            vs = list(rc)
            for j in range(D // 16):
                sl = pl.ds(j * 16, 16)
                e = gd_v[r, sl] + geb_v[r, sl] + msg_v[r, sl]
                gd_v[r, sl] = e
                sig = 1.0 / (1.0 + jnp.exp(-e))
                msg_v[r, sl] = sig * geb_v[r, pl.ds(D + j * 16, 16)]
                vs[j] = vs[j] + e
                vs[8 + j] = vs[8 + j] + e * e
            return tuple(vs)

        carry = lax.fori_loop(0, K, _row, carry)
        pltpu.sync_copy(gd_v, eij_hbm.at[pl.ds(base, K)])
        pltpu.sync_copy(msg_v, aggr_sp.at[row_v], add=True)
        return carry

    zeros = tuple(jnp.zeros((16,), jnp.float32) for _ in range(16))
    sums = lax.fori_loop(0, NCHUNK, _chunk, zeros)

    for j in range(D // 16):
        st_v[0, pl.ds(j * 16, 16)] = sums[j]
        st_v[1, pl.ds(j * 16, 16)] = sums[8 + j]
    pltpu.sync_copy(st_v, stats_hbm.at[wid])

    plsc.subcore_barrier()

    def _out_blk(t, _):
        pltpu.sync_copy(aggr_sp.at[pl.ds(base_row + t * K, K)],
                        aggr_hbm.at[c, pl.ds(base_row + t * K, K)])
        return 0

    lax.fori_loop(0, nblk, _out_blk, 0)


def _node_body(x_ref, ax_ref, agg_ref, g_ref, b_ref, out_ref):
    t = ax_ref[...] + agg_ref[0] + agg_ref[1]
    m = jnp.mean(t, axis=0, keepdims=True)
    d = t - m
    v = jnp.mean(d * d, axis=0, keepdims=True)
    y = d * lax.rsqrt(v + EPS) * g_ref[...] + b_ref[...]
    out_ref[...] = x_ref[...] + jnp.maximum(y, 0.0)


def _efin_body(eij_ref, attr_ref, rwt_ref, stats_ref, g_ref, b_ref, out_ref):
    ssum = jnp.sum(stats_ref[:, 0, :], axis=0, keepdims=True)
    ssq = jnp.sum(stats_ref[:, 1, :], axis=0, keepdims=True)
    m = ssum * (1.0 / E)
    v = ssq * (1.0 / E) - m * m
    scale = lax.rsqrt(v + EPS) * g_ref[...]
    y = (eij_ref[...] - m) * scale + b_ref[...]
    out_ref[...] = (
        jnp.dot(attr_ref[...], rwt_ref[...], precision=_HIGH)
        + jnp.maximum(y, 0.0)
    )


def kernel(x_in_node, edge_idx, edge_in_attr, A_w, A_b, B_w, B_b, C_w, C_b,
           D_w, D_b, E_w, E_b, bn_x_g, bn_x_b, bn_e_g, bn_e_b, res_e_w):
    row = jnp.asarray(edge_idx[0], jnp.int32)
    col = jnp.asarray(edge_idx[1], jnp.int32)

    BN_BLK = 2000
    wspec = pl.BlockSpec((D, D), lambda i: (0, 0))
    bspec = pl.BlockSpec((1, D), lambda i: (0, 0))
    nspec = pl.BlockSpec((BN_BLK, D), lambda i: (i, 0))
    ax, dtab, ebtab = pl.pallas_call(
        _prep_body,
        grid=(N // BN_BLK,),
        in_specs=[nspec, wspec, bspec, wspec, bspec, wspec, bspec, wspec,
                  bspec],
        out_specs=(nspec, nspec, pl.BlockSpec((BN_BLK, 2 * D),
                                              lambda i: (i, 0))),
        out_shape=(
            jax.ShapeDtypeStruct((N, D), jnp.float32),
            jax.ShapeDtypeStruct((N, D), jnp.float32),
            jax.ShapeDtypeStruct((N, 2 * D), jnp.float32),
        ),
    )(x_in_node, A_w.T, A_b[None, :], B_w.T, B_b[None, :], D_w.T,
      D_b[None, :], E_w.T, E_b[None, :])

    BE = 4000
    ce = pl.pallas_call(
        _ce_body,
        grid=(E // BE,),
        in_specs=[
            pl.BlockSpec((BE, DE), lambda i: (i, 0)),
            pl.BlockSpec((DE, D), lambda i: (0, 0)),
            pl.BlockSpec((1, D), lambda i: (0, 0)),
        ],
        out_specs=pl.BlockSpec((BE, D), lambda i: (i, 0)),
        out_shape=jax.ShapeDtypeStruct((E, D), jnp.float32),
    )(edge_in_attr, C_w.T, C_b[None, :])

    sc_edge = functools.partial(
        pl.kernel,
        out_type=(
            jax.ShapeDtypeStruct((E, D), jnp.float32),
            jax.ShapeDtypeStruct((NC, N, D), jnp.float32),
            jax.ShapeDtypeStruct((NW, 2, D), jnp.float32),
        ),
        mesh=plsc.VectorSubcoreMesh(core_axis_name="c", subcore_axis_name="s"),
        scratch_types=[
            pltpu.VMEM((K,), jnp.int32),
            pltpu.VMEM((K,), jnp.int32),
            pltpu.VMEM((K, D), jnp.float32),
            pltpu.VMEM((K, 2 * D), jnp.float32),
            pltpu.VMEM((K, D), jnp.float32),
            pltpu.VMEM((2, D), jnp.float32),
            pltpu.VMEM_SHARED((N, D), jnp.float32),
            pltpu.SemaphoreType.DMA,
            pltpu.SemaphoreType.DMA,
            pltpu.SemaphoreType.DMA,
        ],
    )(_sc_edge_body)
    eij, aggr, stats = sc_edge(row, col, dtab, ebtab, ce)

    x_final = pl.pallas_call(
        _node_body,
        out_shape=jax.ShapeDtypeStruct((N, D), jnp.float32),
    )(x_in_node, ax, aggr, bn_x_g[None, :], bn_x_b[None, :])

    BF = 4000
    e_final = pl.pallas_call(
        _efin_body,
        grid=(E // BF,),
        in_specs=[
            pl.BlockSpec((BF, D), lambda i: (i, 0)),
            pl.BlockSpec((BF, DE), lambda i: (i, 0)),
            pl.BlockSpec((DE, D), lambda i: (0, 0)),
            pl.BlockSpec((NW, 2, D), lambda i: (0, 0, 0)),
            pl.BlockSpec((1, D), lambda i: (0, 0)),
            pl.BlockSpec((1, D), lambda i: (0, 0)),
        ],
        out_specs=pl.BlockSpec((BF, D), lambda i: (i, 0)),
        out_shape=jax.ShapeDtypeStruct((E, D), jnp.float32),
    )(eij, edge_in_attr, res_e_w.T, stats, bn_e_g[None, :], bn_e_b[None, :])

    return (x_final, e_final)


# speedup vs baseline: 3.1781x; 2.5900x over previous
import functools

import jax
import jax.numpy as jnp
from jax import lax
from jax.experimental import pallas as pl
from jax.experimental.pallas import tpu as pltpu
from jax.experimental.pallas import tpu_sc as plsc

N = 10000
E = 320000
D = 128
DE = 16
EPS = 1e-5

NC = 2
NS = 16
NW = NC * NS
EPW = E // NW
K = 40
NCHUNK = EPW // K
ROWS_PT = 640

_HIGH = lax.Precision.HIGHEST


def _prep_body(x_ref, awt, ab, bwt, bb, dwt, db, ewt, eb_, ax_out, dtab_out,
               etab_out, btab_out):
    x = x_ref[...]
    ax_out[...] = jnp.dot(x, awt[...], precision=_HIGH) + ab[...]
    dtab_out[...] = jnp.dot(x, dwt[...], precision=_HIGH) + db[...]
    etab_out[...] = jnp.dot(x, ewt[...], precision=_HIGH) + eb_[...]
    btab_out[...] = jnp.dot(x, bwt[...], precision=_HIGH) + bb[...]


def _ce_body(attr_ref, cwt_ref, cb_ref, ce_out):
    ce_out[...] = (
        jnp.dot(attr_ref[...], cwt_ref[...], precision=_HIGH) + cb_ref[...]
    )


def _sc_edge_body(row_hbm, col_hbm, dtab_hbm, etab_hbm, btab_hbm, ce_hbm,
                  eij_hbm, aggr_hbm, stats_hbm,
                  row0, col0, gd0, ge0, gb0, ms0,
                  row1, col1, gd1, ge1, gb1, ms1,
                  st_v, aggr_sp, sem0, sem1):
    c = lax.axis_index("c")
    s = lax.axis_index("s")
    wid = c * NS + s
    rows = (row0, row1)
    cols = (col0, col1)
    gds = (gd0, gd1)
    ges = (ge0, ge1)
    gbs = (gb0, gb1)
    mss = (ms0, ms1)
    sems = (sem0, sem1)

    def _zero_row(r, _):
        for j in range(D // 16):
            ms0[r, pl.ds(j * 16, 16)] = jnp.zeros((16,), jnp.float32)
        return 0

    lax.fori_loop(0, K, _zero_row, 0)
    base_row = s * ROWS_PT
    nblk = jnp.where(s == NS - 1, (N - (NS - 1) * ROWS_PT) // K,
                     ROWS_PT // K)

    def _zero_blk(t, _):
        pltpu.sync_copy(ms0, aggr_sp.at[pl.ds(base_row + t * K, K)])
        return 0

    lax.fori_loop(0, nblk, _zero_blk, 0)
    plsc.subcore_barrier()

    e0 = wid * EPW

    def _issue(b, ci):
        base = e0 + ci * K
        pltpu.sync_copy(row_hbm.at[pl.ds(base, K)], rows[b])
        pltpu.sync_copy(col_hbm.at[pl.ds(base, K)], cols[b])
        pltpu.async_copy(dtab_hbm.at[rows[b]], gds[b], sems[b])
        pltpu.async_copy(etab_hbm.at[cols[b]], ges[b], sems[b])
        pltpu.async_copy(btab_hbm.at[cols[b]], gbs[b], sems[b])
        pltpu.async_copy(ce_hbm.at[pl.ds(base, K)], mss[b], sems[b])

    def _wait(b):
        pltpu.make_async_copy(dtab_hbm.at[rows[b]], gds[b], sems[b]).wait()
        pltpu.make_async_copy(etab_hbm.at[cols[b]], ges[b], sems[b]).wait()
        pltpu.make_async_copy(btab_hbm.at[cols[b]], gbs[b], sems[b]).wait()
        pltpu.make_async_copy(ce_hbm.at[pl.ds(0, K)], mss[b], sems[b]).wait()

    def _compute(b, carry):
        gd_v, ge_v, gb_v, ms_v = gds[b], ges[b], gbs[b], mss[b]

        def _row(r, rc):
            vs = list(rc)
            for j in range(D // 16):
                sl = pl.ds(j * 16, 16)
                e = gd_v[r, sl] + ge_v[r, sl] + ms_v[r, sl]
                gd_v[r, sl] = e
                sig = 1.0 / (1.0 + jnp.exp(-e))
                ms_v[r, sl] = sig * gb_v[r, sl]
                vs[j] = vs[j] + e
                vs[8 + j] = vs[8 + j] + e * e
            return tuple(vs)

        return lax.fori_loop(0, K, _row, carry)

    _issue(0, 0)

    def _step(step, carry):
        for b in range(2):
            ci = 2 * step + b

            @pl.when(ci + 1 < NCHUNK)
            def _():
                _issue(1 - b, ci + 1)

            _wait(b)
            carry = _compute(b, carry)
            base = e0 + ci * K
            pltpu.sync_copy(gds[b], eij_hbm.at[pl.ds(base, K)])
            pltpu.sync_copy(mss[b], aggr_sp.at[rows[b]], add=True)
        return carry

    zeros = tuple(jnp.zeros((16,), jnp.float32) for _ in range(16))
    sums = lax.fori_loop(0, NCHUNK // 2, _step, zeros)

    for j in range(D // 16):
        st_v[0, pl.ds(j * 16, 16)] = sums[j]
        st_v[1, pl.ds(j * 16, 16)] = sums[8 + j]
    pltpu.sync_copy(st_v, stats_hbm.at[wid])

    plsc.subcore_barrier()

    def _out_blk(t, _):
        pltpu.sync_copy(aggr_sp.at[pl.ds(base_row + t * K, K)],
                        aggr_hbm.at[c, pl.ds(base_row + t * K, K)])
        return 0

    lax.fori_loop(0, nblk, _out_blk, 0)


def _node_body(x_ref, ax_ref, agg_ref, g_ref, b_ref, out_ref):
    t = ax_ref[...] + agg_ref[0] + agg_ref[1]
    m = jnp.mean(t, axis=0, keepdims=True)
    d = t - m
    v = jnp.mean(d * d, axis=0, keepdims=True)
    y = d * lax.rsqrt(v + EPS) * g_ref[...] + b_ref[...]
    out_ref[...] = x_ref[...] + jnp.maximum(y, 0.0)


def _efin_body(eij_ref, attr_ref, rwt_ref, stats_ref, g_ref, b_ref, out_ref):
    ssum = jnp.sum(stats_ref[:, 0, :], axis=0, keepdims=True)
    ssq = jnp.sum(stats_ref[:, 1, :], axis=0, keepdims=True)
    m = ssum * (1.0 / E)
    v = ssq * (1.0 / E) - m * m
    scale = lax.rsqrt(v + EPS) * g_ref[...]
    y = (eij_ref[...] - m) * scale + b_ref[...]
    out_ref[...] = (
        jnp.dot(attr_ref[...], rwt_ref[...], precision=_HIGH)
        + jnp.maximum(y, 0.0)
    )


def kernel(x_in_node, edge_idx, edge_in_attr, A_w, A_b, B_w, B_b, C_w, C_b,
           D_w, D_b, E_w, E_b, bn_x_g, bn_x_b, bn_e_g, bn_e_b, res_e_w):
    row = jnp.asarray(edge_idx[0], jnp.int32)
    col = jnp.asarray(edge_idx[1], jnp.int32)

    BN_BLK = 2000
    wspec = pl.BlockSpec((D, D), lambda i: (0, 0))
    bspec = pl.BlockSpec((1, D), lambda i: (0, 0))
    nspec = pl.BlockSpec((BN_BLK, D), lambda i: (i, 0))
    ax, dtab, etab, btab = pl.pallas_call(
        _prep_body,
        grid=(N // BN_BLK,),
        in_specs=[nspec, wspec, bspec, wspec, bspec, wspec, bspec, wspec,
                  bspec],
        out_specs=(nspec, nspec, nspec, nspec),
        out_shape=(
            jax.ShapeDtypeStruct((N, D), jnp.float32),
            jax.ShapeDtypeStruct((N, D), jnp.float32),
            jax.ShapeDtypeStruct((N, D), jnp.float32),
            jax.ShapeDtypeStruct((N, D), jnp.float32),
        ),
    )(x_in_node, A_w.T, A_b[None, :], B_w.T, B_b[None, :], D_w.T,
      D_b[None, :], E_w.T, E_b[None, :])

    BE = 4000
    ce = pl.pallas_call(
        _ce_body,
        grid=(E // BE,),
        in_specs=[
            pl.BlockSpec((BE, DE), lambda i: (i, 0)),
            pl.BlockSpec((DE, D), lambda i: (0, 0)),
            pl.BlockSpec((1, D), lambda i: (0, 0)),
        ],
        out_specs=pl.BlockSpec((BE, D), lambda i: (i, 0)),
        out_shape=jax.ShapeDtypeStruct((E, D), jnp.float32),
    )(edge_in_attr, C_w.T, C_b[None, :])

    sc_edge = functools.partial(
        pl.kernel,
        out_type=(
            jax.ShapeDtypeStruct((E, D), jnp.float32),
            jax.ShapeDtypeStruct((NC, N, D), jnp.float32),
            jax.ShapeDtypeStruct((NW, 2, D), jnp.float32),
        ),
        mesh=plsc.VectorSubcoreMesh(core_axis_name="c", subcore_axis_name="s"),
        scratch_types=[
            pltpu.VMEM((K,), jnp.int32),
            pltpu.VMEM((K,), jnp.int32),
            pltpu.VMEM((K, D), jnp.float32),
            pltpu.VMEM((K, D), jnp.float32),
            pltpu.VMEM((K, D), jnp.float32),
            pltpu.VMEM((K, D), jnp.float32),
            pltpu.VMEM((K,), jnp.int32),
            pltpu.VMEM((K,), jnp.int32),
            pltpu.VMEM((K, D), jnp.float32),
            pltpu.VMEM((K, D), jnp.float32),
            pltpu.VMEM((K, D), jnp.float32),
            pltpu.VMEM((K, D), jnp.float32),
            pltpu.VMEM((2, D), jnp.float32),
            pltpu.VMEM_SHARED((N, D), jnp.float32),
            pltpu.SemaphoreType.DMA,
            pltpu.SemaphoreType.DMA,
        ],
    )(_sc_edge_body)
    eij, aggr, stats = sc_edge(row, col, dtab, etab, btab, ce)

    x_final = pl.pallas_call(
        _node_body,
        out_shape=jax.ShapeDtypeStruct((N, D), jnp.float32),
    )(x_in_node, ax, aggr, bn_x_g[None, :], bn_x_b[None, :])

    BF = 4000
    e_final = pl.pallas_call(
        _efin_body,
        grid=(E // BF,),
        in_specs=[
            pl.BlockSpec((BF, D), lambda i: (i, 0)),
            pl.BlockSpec((BF, DE), lambda i: (i, 0)),
            pl.BlockSpec((DE, D), lambda i: (0, 0)),
            pl.BlockSpec((NW, 2, D), lambda i: (0, 0, 0)),
            pl.BlockSpec((1, D), lambda i: (0, 0)),
            pl.BlockSpec((1, D), lambda i: (0, 0)),
        ],
        out_specs=pl.BlockSpec((BF, D), lambda i: (i, 0)),
        out_shape=jax.ShapeDtypeStruct((E, D), jnp.float32),
    )(eij, edge_in_attr, res_e_w.T, stats, bn_e_g[None, :], bn_e_b[None, :])

    return (x_final, e_final)


# speedup vs baseline: 3.3267x; 1.0467x over previous
import functools

import jax
import jax.numpy as jnp
from jax import lax
from jax.experimental import pallas as pl
from jax.experimental.pallas import tpu as pltpu
from jax.experimental.pallas import tpu_sc as plsc

N = 10000
E = 320000
D = 128
DE = 16
EPS = 1e-5

NC = 2
NS = 16
NW = NC * NS
EPW = E // NW
K = 40
NCHUNK = EPW // K
ROWS_PT = 640

_HIGH = lax.Precision.HIGHEST
_MED = lax.Precision.DEFAULT
_ILV = plsc.PackFormat.INTERLEAVED


def _pack_tc(lo_f32, hi_f32):
    lo = lax.bitcast_convert_type(lo_f32.astype(jnp.bfloat16),
                                  jnp.uint16).astype(jnp.uint32)
    hi = lax.bitcast_convert_type(hi_f32.astype(jnp.bfloat16),
                                  jnp.uint16).astype(jnp.uint32)
    return lax.bitcast_convert_type(lo | (hi << 16), jnp.float32)


def _unpack_tc(w_f32):
    u = lax.bitcast_convert_type(w_f32, jnp.uint32)
    lo = lax.bitcast_convert_type((u & 0xFFFF).astype(jnp.uint16),
                                  jnp.bfloat16).astype(jnp.float32)
    hi = lax.bitcast_convert_type((u >> 16).astype(jnp.uint16),
                                  jnp.bfloat16).astype(jnp.float32)
    return lo, hi


def _prep_body(x_ref, awt, ab, bwt, bb, dwt, db, ewt, eb_, ax_out, dtab_out,
               ebtab_out):
    x = x_ref[...]
    ax_out[...] = jnp.dot(x, awt[...], precision=_HIGH) + ab[...]
    dtab_out[...] = jnp.dot(x, dwt[...], precision=_HIGH) + db[...]
    ex = jnp.dot(x, ewt[...], precision=_HIGH) + eb_[...]
    bx = jnp.dot(x, bwt[...], precision=_HIGH) + bb[...]
    ebtab_out[...] = _pack_tc(ex, bx)


def _ce_body(attr_ref, cwt_ref, cb_ref, ce_out):
    cv = jnp.dot(attr_ref[...], cwt_ref[...], precision=_MED) + cb_ref[...]
    ce_out[...] = _pack_tc(cv[:, : D // 2], cv[:, D // 2:])


def _sc_edge_body(row_hbm, col_hbm, dtab_hbm, ebtab_hbm, ce_hbm,
                  eij_hbm, aggr_hbm, stats_hbm,
                  row0, col0, gd0, geb0,
                  row1, col1, gd1, geb1,
                  ce_v, eij_v, msg_v, st_v, aggr_sp, sem0, sem1, sem_ce):
    c = lax.axis_index("c")
    s = lax.axis_index("s")
    wid = c * NS + s
    rows = (row0, row1)
    cols = (col0, col1)
    gds = (gd0, gd1)
    gebs = (geb0, geb1)
    sems = (sem0, sem1)

    def _zero_row(r, _):
        for j in range(D // 16):
            msg_v[r, pl.ds(j * 16, 16)] = jnp.zeros((16,), jnp.float32)
        return 0

    lax.fori_loop(0, K, _zero_row, 0)
    base_row = s * ROWS_PT
    nblk = jnp.where(s == NS - 1, (N - (NS - 1) * ROWS_PT) // K,
                     ROWS_PT // K)

    def _zero_blk(t, _):
        pltpu.sync_copy(msg_v, aggr_sp.at[pl.ds(base_row + t * K, K)])
        return 0

    lax.fori_loop(0, nblk, _zero_blk, 0)
    plsc.subcore_barrier()

    e0 = wid * EPW

    def _issue(b, ci):
        base = e0 + ci * K
        pltpu.sync_copy(row_hbm.at[pl.ds(base, K)], rows[b])
        pltpu.sync_copy(col_hbm.at[pl.ds(base, K)], cols[b])
        pltpu.async_copy(dtab_hbm.at[rows[b]], gds[b], sems[b])
        pltpu.async_copy(ebtab_hbm.at[cols[b]], gebs[b], sems[b])

    def _issue_ce(ci):
        base = e0 + ci * K
        pltpu.async_copy(ce_hbm.at[pl.ds(base, K)], ce_v, sem_ce)

    def _wait(b):
        pltpu.make_async_copy(dtab_hbm.at[rows[b]], gds[b], sems[b]).wait()
        pltpu.make_async_copy(ebtab_hbm.at[cols[b]], gebs[b], sems[b]).wait()
        pltpu.make_async_copy(ce_hbm.at[pl.ds(0, K)], ce_v, sem_ce).wait()

    def _compute(b, carry):
        gd_v, geb_v = gds[b], gebs[b]

        def _row(r, rc):
            vs = list(rc)
            for j in range(D // 32):
                lo16 = pl.ds(j * 16, 16)
                hi16 = pl.ds(64 + j * 16, 16)
                c_lo, c_hi = plsc.unpack(
                    plsc.bitcast(ce_v[r, lo16], jnp.bfloat16), format=_ILV)
                e1, b1 = plsc.unpack(
                    plsc.bitcast(geb_v[r, lo16], jnp.bfloat16), format=_ILV)
                e2, b2 = plsc.unpack(
                    plsc.bitcast(geb_v[r, hi16], jnp.bfloat16), format=_ILV)
                elo = gd_v[r, lo16] + e1 + c_lo
                ehi = gd_v[r, hi16] + e2 + c_hi
                eij_v[r, lo16] = plsc.bitcast(
                    plsc.pack(elo, ehi, format=_ILV), jnp.float32)
                siglo = 1.0 / (1.0 + jnp.exp(-elo))
                sighi = 1.0 / (1.0 + jnp.exp(-ehi))
                msg_v[r, lo16] = siglo * b1
                msg_v[r, hi16] = sighi * b2
                vs[j] = vs[j] + elo
                vs[4 + j] = vs[4 + j] + ehi
                vs[8 + j] = vs[8 + j] + elo * elo
                vs[12 + j] = vs[12 + j] + ehi * ehi
            return tuple(vs)

        return lax.fori_loop(0, K, _row, carry)

    _issue(0, 0)
    _issue_ce(0)

    def _step(step, carry):
        for b in range(2):
            ci = 2 * step + b

            @pl.when(ci + 1 < NCHUNK)
            def _():
                _issue(1 - b, ci + 1)

            _wait(b)
            carry = _compute(b, carry)

            @pl.when(ci + 1 < NCHUNK)
            def _():
                _issue_ce(ci + 1)

            base = e0 + ci * K
            pltpu.sync_copy(eij_v, eij_hbm.at[pl.ds(base, K)])
            pltpu.sync_copy(msg_v, aggr_sp.at[rows[b]], add=True)
        return carry

    zeros = tuple(jnp.zeros((16,), jnp.float32) for _ in range(16))
    sums = lax.fori_loop(0, NCHUNK // 2, _step, zeros)

    for j in range(D // 32):
        st_v[0, pl.ds(j * 16, 16)] = sums[j]
        st_v[0, pl.ds(64 + j * 16, 16)] = sums[4 + j]
        st_v[1, pl.ds(j * 16, 16)] = sums[8 + j]
        st_v[1, pl.ds(64 + j * 16, 16)] = sums[12 + j]
    pltpu.sync_copy(st_v, stats_hbm.at[wid])

    plsc.subcore_barrier()

    def _out_blk(t, _):
        pltpu.sync_copy(aggr_sp.at[pl.ds(base_row + t * K, K)],
                        aggr_hbm.at[c, pl.ds(base_row + t * K, K)])
        return 0

    lax.fori_loop(0, nblk, _out_blk, 0)


def _node_body(x_ref, ax_ref, agg_ref, g_ref, b_ref, out_ref):
    t = ax_ref[...] + agg_ref[0] + agg_ref[1]
    m = jnp.mean(t, axis=0, keepdims=True)
    d = t - m
    v = jnp.mean(d * d, axis=0, keepdims=True)
    y = d * lax.rsqrt(v + EPS) * g_ref[...] + b_ref[...]
    out_ref[...] = x_ref[...] + jnp.maximum(y, 0.0)


def _efin_body(eij_ref, attr_ref, rwt_ref, stats_ref, g_ref, b_ref, out_ref):
    ssum = jnp.sum(stats_ref[:, 0, :], axis=0, keepdims=True)
    ssq = jnp.sum(stats_ref[:, 1, :], axis=0, keepdims=True)
    m = ssum * (1.0 / E)
    v = ssq * (1.0 / E) - m * m
    scale = lax.rsqrt(v + EPS) * g_ref[...]
    elo, ehi = _unpack_tc(eij_ref[...])
    res = jnp.dot(attr_ref[...], rwt_ref[...], precision=_MED)
    h = D // 2
    ylo = (elo - m[:, :h]) * scale[:, :h] + b_ref[:, :h]
    yhi = (ehi - m[:, h:]) * scale[:, h:] + b_ref[:, h:]
    out_ref[:, :h] = res[:, :h] + jnp.maximum(ylo, 0.0)
    out_ref[:, h:] = res[:, h:] + jnp.maximum(yhi, 0.0)


def kernel(x_in_node, edge_idx, edge_in_attr, A_w, A_b, B_w, B_b, C_w, C_b,
           D_w, D_b, E_w, E_b, bn_x_g, bn_x_b, bn_e_g, bn_e_b, res_e_w):
    row = jnp.asarray(edge_idx[0], jnp.int32)
    col = jnp.asarray(edge_idx[1], jnp.int32)

    BN_BLK = 2000
    wspec = pl.BlockSpec((D, D), lambda i: (0, 0))
    bspec = pl.BlockSpec((1, D), lambda i: (0, 0))
    nspec = pl.BlockSpec((BN_BLK, D), lambda i: (i, 0))
    ax, dtab, ebtab = pl.pallas_call(
        _prep_body,
        grid=(N // BN_BLK,),
        in_specs=[nspec, wspec, bspec, wspec, bspec, wspec, bspec, wspec,
                  bspec],
        out_specs=(nspec, nspec, nspec),
        out_shape=(
            jax.ShapeDtypeStruct((N, D), jnp.float32),
            jax.ShapeDtypeStruct((N, D), jnp.float32),
            jax.ShapeDtypeStruct((N, D), jnp.float32),
        ),
    )(x_in_node, A_w.T, A_b[None, :], B_w.T, B_b[None, :], D_w.T,
      D_b[None, :], E_w.T, E_b[None, :])

    BE = 4000
    ce = pl.pallas_call(
        _ce_body,
        grid=(E // BE,),
        in_specs=[
            pl.BlockSpec((BE, DE), lambda i: (i, 0)),
            pl.BlockSpec((DE, D), lambda i: (0, 0)),
            pl.BlockSpec((1, D), lambda i: (0, 0)),
        ],
        out_specs=pl.BlockSpec((BE, D // 2), lambda i: (i, 0)),
        out_shape=jax.ShapeDtypeStruct((E, D // 2), jnp.float32),
    )(edge_in_attr, C_w.T, C_b[None, :])

    sc_edge = functools.partial(
        pl.kernel,
        out_type=(
            jax.ShapeDtypeStruct((E, D // 2), jnp.float32),
            jax.ShapeDtypeStruct((NC, N, D), jnp.float32),
            jax.ShapeDtypeStruct((NW, 2, D), jnp.float32),
        ),
        mesh=plsc.VectorSubcoreMesh(core_axis_name="c", subcore_axis_name="s"),
        compiler_params=pltpu.CompilerParams(needs_layout_passes=False),
        scratch_types=[
            pltpu.VMEM((K,), jnp.int32),
            pltpu.VMEM((K,), jnp.int32),
            pltpu.VMEM((K, D), jnp.float32),
            pltpu.VMEM((K, D), jnp.float32),
            pltpu.VMEM((K,), jnp.int32),
            pltpu.VMEM((K,), jnp.int32),
            pltpu.VMEM((K, D), jnp.float32),
            pltpu.VMEM((K, D), jnp.float32),
            pltpu.VMEM((K, D // 2), jnp.float32),
            pltpu.VMEM((K, D // 2), jnp.float32),
            pltpu.VMEM((K, D), jnp.float32),
            pltpu.VMEM((2, D), jnp.float32),
            pltpu.VMEM_SHARED((N, D), jnp.float32),
            pltpu.SemaphoreType.DMA,
            pltpu.SemaphoreType.DMA,
            pltpu.SemaphoreType.DMA,
        ],
    )(_sc_edge_body)
    eij, aggr, stats = sc_edge(row, col, dtab, ebtab, ce)

    x_final = pl.pallas_call(
        _node_body,
        out_shape=jax.ShapeDtypeStruct((N, D), jnp.float32),
    )(x_in_node, ax, aggr, bn_x_g[None, :], bn_x_b[None, :])

    BF = 4000
    e_final = pl.pallas_call(
        _efin_body,
        grid=(E // BF,),
        in_specs=[
            pl.BlockSpec((BF, D // 2), lambda i: (i, 0)),
            pl.BlockSpec((BF, DE), lambda i: (i, 0)),
            pl.BlockSpec((DE, D), lambda i: (0, 0)),
            pl.BlockSpec((NW, 2, D), lambda i: (0, 0, 0)),
            pl.BlockSpec((1, D), lambda i: (0, 0)),
            pl.BlockSpec((1, D), lambda i: (0, 0)),
        ],
        out_specs=pl.BlockSpec((BF, D), lambda i: (i, 0)),
        out_shape=jax.ShapeDtypeStruct((E, D), jnp.float32),
    )(eij, edge_in_attr, res_e_w.T, stats, bn_e_g[None, :], bn_e_b[None, :])

    return (x_final, e_final)


# speedup vs baseline: 3.4503x; 1.0372x over previous
import functools

import jax
import jax.numpy as jnp
from jax import lax
from jax.experimental import pallas as pl
from jax.experimental.pallas import tpu as pltpu
from jax.experimental.pallas import tpu_sc as plsc

N = 10000
E = 320000
D = 128
DE = 16
EPS = 1e-5

NC = 2
NS = 16
NW = NC * NS
EPW = E // NW
K = 40
NCHUNK = EPW // K
ROWS_PT = 640

_HIGH = lax.Precision.HIGHEST
_MED = lax.Precision.DEFAULT
_ILV = plsc.PackFormat.INTERLEAVED


def _pack_tc(lo_f32, hi_f32):
    lo = lax.bitcast_convert_type(lo_f32.astype(jnp.bfloat16),
                                  jnp.uint16).astype(jnp.uint32)
    hi = lax.bitcast_convert_type(hi_f32.astype(jnp.bfloat16),
                                  jnp.uint16).astype(jnp.uint32)
    return lax.bitcast_convert_type(lo | (hi << 16), jnp.float32)


def _unpack_tc(w_f32):
    u = lax.bitcast_convert_type(w_f32, jnp.uint32)
    lo = lax.bitcast_convert_type((u & 0xFFFF).astype(jnp.uint16),
                                  jnp.bfloat16).astype(jnp.float32)
    hi = lax.bitcast_convert_type((u >> 16).astype(jnp.uint16),
                                  jnp.bfloat16).astype(jnp.float32)
    return lo, hi


def _prep_body(x_ref, awt, ab, bwt, bb, dwt, db, ewt, eb_, ax_out, dtab_out,
               ebtab_out):
    x = x_ref[...]
    ax_out[...] = jnp.dot(x, awt[...], precision=_HIGH) + ab[...]
    dtab_out[...] = jnp.dot(x, dwt[...], precision=_HIGH) + db[...]
    ex = jnp.dot(x, ewt[...], precision=_HIGH) + eb_[...]
    bx = jnp.dot(x, bwt[...], precision=_HIGH) + bb[...]
    ebtab_out[...] = _pack_tc(ex, bx)


def _ce_body(attr_ref, cwt_ref, cb_ref, ce_out):
    cv = jnp.dot(attr_ref[...], cwt_ref[...], precision=_MED) + cb_ref[...]
    ce_out[...] = _pack_tc(cv[:, : D // 2], cv[:, D // 2:])


def _sc_edge_body(row_hbm, col_hbm, dtab_hbm, ebtab_hbm, ce_hbm,
                  eij_hbm, aggr_hbm, stats_hbm,
                  row0, col0, gd0, geb0, eij0,
                  row1, col1, gd1, geb1, eij1,
                  ce_v, msg_v, st_v, aggr_sp, sem0, sem1, sem_ce,
                  sem_eij0, sem_eij1):
    c = lax.axis_index("c")
    s = lax.axis_index("s")
    wid = c * NS + s
    rows = (row0, row1)
    cols = (col0, col1)
    gds = (gd0, gd1)
    gebs = (geb0, geb1)
    eijs = (eij0, eij1)
    sems = (sem0, sem1)
    sems_eij = (sem_eij0, sem_eij1)

    def _zero_row(r, _):
        for j in range(D // 16):
            msg_v[r, pl.ds(j * 16, 16)] = jnp.zeros((16,), jnp.float32)
        return 0

    lax.fori_loop(0, K, _zero_row, 0)
    base_row = s * ROWS_PT
    nblk = jnp.where(s == NS - 1, (N - (NS - 1) * ROWS_PT) // K,
                     ROWS_PT // K)

    def _zero_blk(t, _):
        pltpu.sync_copy(msg_v, aggr_sp.at[pl.ds(base_row + t * K, K)])
        return 0

    lax.fori_loop(0, nblk, _zero_blk, 0)
    plsc.subcore_barrier()

    e0 = wid * EPW

    def _issue(b, ci):
        base = e0 + ci * K
        pltpu.sync_copy(row_hbm.at[pl.ds(base, K)], rows[b])
        pltpu.sync_copy(col_hbm.at[pl.ds(base, K)], cols[b])
        pltpu.async_copy(dtab_hbm.at[rows[b]], gds[b], sems[b])
        pltpu.async_copy(ebtab_hbm.at[cols[b]], gebs[b], sems[b])

    def _issue_ce(ci):
        base = e0 + ci * K
        pltpu.async_copy(ce_hbm.at[pl.ds(base, K)], ce_v, sem_ce)

    def _wait(b):
        pltpu.make_async_copy(dtab_hbm.at[rows[b]], gds[b], sems[b]).wait()
        pltpu.make_async_copy(ebtab_hbm.at[cols[b]], gebs[b],
                              sems[b]).wait()
        pltpu.make_async_copy(ce_hbm.at[pl.ds(0, K)], ce_v, sem_ce).wait()

    def _wait_eij(b):
        pltpu.make_async_copy(eijs[b], eij_hbm.at[pl.ds(0, K)],
                              sems_eij[b]).wait()

    def _compute(b, carry):
        gd_v, geb_v, eij_v = gds[b], gebs[b], eijs[b]

        def _row(r, rc):
            vs = list(rc)
            for j in range(D // 32):
                lo16 = pl.ds(j * 16, 16)
                hi16 = pl.ds(64 + j * 16, 16)
                c_lo, c_hi = plsc.unpack(
                    plsc.bitcast(ce_v[r, lo16], jnp.bfloat16), format=_ILV)
                e1, b1 = plsc.unpack(
                    plsc.bitcast(geb_v[r, lo16], jnp.bfloat16), format=_ILV)
                e2, b2 = plsc.unpack(
                    plsc.bitcast(geb_v[r, hi16], jnp.bfloat16), format=_ILV)
                elo = gd_v[r, lo16] + e1 + c_lo
                ehi = gd_v[r, hi16] + e2 + c_hi
                eij_v[r, lo16] = plsc.bitcast(
                    plsc.pack(elo, ehi, format=_ILV), jnp.float32)
                siglo = 1.0 / (1.0 + jnp.exp(-elo))
                sighi = 1.0 / (1.0 + jnp.exp(-ehi))
                msg_v[r, lo16] = siglo * b1
                msg_v[r, hi16] = sighi * b2
                vs[j] = vs[j] + elo
                vs[4 + j] = vs[4 + j] + ehi
                vs[8 + j] = vs[8 + j] + elo * elo
                vs[12 + j] = vs[12 + j] + ehi * ehi
            return tuple(vs)

        return lax.fori_loop(0, K, _row, carry)

    _issue(0, 0)
    _issue_ce(0)

    def _step(step, carry):
        for b in range(2):
            ci = 2 * step + b

            @pl.when(ci + 1 < NCHUNK)
            def _():
                _issue(1 - b, ci + 1)

            _wait(b)

            @pl.when(ci >= 2)
            def _():
                _wait_eij(b)

            carry = _compute(b, carry)

            @pl.when(ci + 1 < NCHUNK)
            def _():
                _issue_ce(ci + 1)

            base = e0 + ci * K
            pltpu.async_copy(eijs[b], eij_hbm.at[pl.ds(base, K)],
                             sems_eij[b])
            pltpu.sync_copy(msg_v, aggr_sp.at[rows[b]], add=True)
        return carry

    zeros = tuple(jnp.zeros((16,), jnp.float32) for _ in range(16))
    sums = lax.fori_loop(0, NCHUNK // 2, _step, zeros)
    _wait_eij(0)
    _wait_eij(1)

    for j in range(D // 32):
        st_v[0, pl.ds(j * 16, 16)] = sums[j]
        st_v[0, pl.ds(64 + j * 16, 16)] = sums[4 + j]
        st_v[1, pl.ds(j * 16, 16)] = sums[8 + j]
        st_v[1, pl.ds(64 + j * 16, 16)] = sums[12 + j]
    pltpu.sync_copy(st_v, stats_hbm.at[wid])

    plsc.subcore_barrier()

    def _out_blk(t, _):
        pltpu.sync_copy(aggr_sp.at[pl.ds(base_row + t * K, K)],
                        aggr_hbm.at[c, pl.ds(base_row + t * K, K)])
        return 0

    lax.fori_loop(0, nblk, _out_blk, 0)


def _node_body(x_ref, ax_ref, agg_ref, g_ref, b_ref, out_ref):
    t = ax_ref[...] + agg_ref[0] + agg_ref[1]
    m = jnp.mean(t, axis=0, keepdims=True)
    d = t - m
    v = jnp.mean(d * d, axis=0, keepdims=True)
    y = d * lax.rsqrt(v + EPS) * g_ref[...] + b_ref[...]
    out_ref[...] = x_ref[...] + jnp.maximum(y, 0.0)


def _efin_body(eij_ref, attr_ref, rwt_ref, stats_ref, g_ref, b_ref, out_ref):
    ssum = jnp.sum(stats_ref[:, 0, :], axis=0, keepdims=True)
    ssq = jnp.sum(stats_ref[:, 1, :], axis=0, keepdims=True)
    m = ssum * (1.0 / E)
    v = ssq * (1.0 / E) - m * m
    scale = lax.rsqrt(v + EPS) * g_ref[...]
    elo, ehi = _unpack_tc(eij_ref[...])
    res = jnp.dot(attr_ref[...], rwt_ref[...], precision=_MED)
    h = D // 2
    ylo = (elo - m[:, :h]) * scale[:, :h] + b_ref[:, :h]
    yhi = (ehi - m[:, h:]) * scale[:, h:] + b_ref[:, h:]
    out_ref[:, :h] = res[:, :h] + jnp.maximum(ylo, 0.0)
    out_ref[:, h:] = res[:, h:] + jnp.maximum(yhi, 0.0)


def kernel(x_in_node, edge_idx, edge_in_attr, A_w, A_b, B_w, B_b, C_w, C_b,
           D_w, D_b, E_w, E_b, bn_x_g, bn_x_b, bn_e_g, bn_e_b, res_e_w):
    row = jnp.asarray(edge_idx[0], jnp.int32)
    col = jnp.asarray(edge_idx[1], jnp.int32)

    BN_BLK = 2000
    wspec = pl.BlockSpec((D, D), lambda i: (0, 0))
    bspec = pl.BlockSpec((1, D), lambda i: (0, 0))
    nspec = pl.BlockSpec((BN_BLK, D), lambda i: (i, 0))
    ax, dtab, ebtab = pl.pallas_call(
        _prep_body,
        grid=(N // BN_BLK,),
        in_specs=[nspec, wspec, bspec, wspec, bspec, wspec, bspec, wspec,
                  bspec],
        out_specs=(nspec, nspec, nspec),
        out_shape=(
            jax.ShapeDtypeStruct((N, D), jnp.float32),
            jax.ShapeDtypeStruct((N, D), jnp.float32),
            jax.ShapeDtypeStruct((N, D), jnp.float32),
        ),
    )(x_in_node, A_w.T, A_b[None, :], B_w.T, B_b[None, :], D_w.T,
      D_b[None, :], E_w.T, E_b[None, :])

    BE = 4000
    ce = pl.pallas_call(
        _ce_body,
        grid=(E // BE,),
        in_specs=[
            pl.BlockSpec((BE, DE), lambda i: (i, 0)),
            pl.BlockSpec((DE, D), lambda i: (0, 0)),
            pl.BlockSpec((1, D), lambda i: (0, 0)),
        ],
        out_specs=pl.BlockSpec((BE, D // 2), lambda i: (i, 0)),
        out_shape=jax.ShapeDtypeStruct((E, D // 2), jnp.float32),
    )(edge_in_attr, C_w.T, C_b[None, :])

    sc_edge = functools.partial(
        pl.kernel,
        out_type=(
            jax.ShapeDtypeStruct((E, D // 2), jnp.float32),
            jax.ShapeDtypeStruct((NC, N, D), jnp.float32),
            jax.ShapeDtypeStruct((NW, 2, D), jnp.float32),
        ),
        mesh=plsc.VectorSubcoreMesh(core_axis_name="c", subcore_axis_name="s"),
        compiler_params=pltpu.CompilerParams(needs_layout_passes=False),
        scratch_types=[
            pltpu.VMEM((K,), jnp.int32),
            pltpu.VMEM((K,), jnp.int32),
            pltpu.VMEM((K, D), jnp.float32),
            pltpu.VMEM((K, D), jnp.float32),
            pltpu.VMEM((K, D // 2), jnp.float32),
            pltpu.VMEM((K,), jnp.int32),
            pltpu.VMEM((K,), jnp.int32),
            pltpu.VMEM((K, D), jnp.float32),
            pltpu.VMEM((K, D), jnp.float32),
            pltpu.VMEM((K, D // 2), jnp.float32),
            pltpu.VMEM((K, D // 2), jnp.float32),
            pltpu.VMEM((K, D), jnp.float32),
            pltpu.VMEM((2, D), jnp.float32),
            pltpu.VMEM_SHARED((N, D), jnp.float32),
            pltpu.SemaphoreType.DMA,
            pltpu.SemaphoreType.DMA,
            pltpu.SemaphoreType.DMA,
            pltpu.SemaphoreType.DMA,
            pltpu.SemaphoreType.DMA,
        ],
    )(_sc_edge_body)
    eij, aggr, stats = sc_edge(row, col, dtab, ebtab, ce)

    x_final = pl.pallas_call(
        _node_body,
        out_shape=jax.ShapeDtypeStruct((N, D), jnp.float32),
    )(x_in_node, ax, aggr, bn_x_g[None, :], bn_x_b[None, :])

    BF = 4000
    e_final = pl.pallas_call(
        _efin_body,
        grid=(E // BF,),
        in_specs=[
            pl.BlockSpec((BF, D // 2), lambda i: (i, 0)),
            pl.BlockSpec((BF, DE), lambda i: (i, 0)),
            pl.BlockSpec((DE, D), lambda i: (0, 0)),
            pl.BlockSpec((NW, 2, D), lambda i: (0, 0, 0)),
            pl.BlockSpec((1, D), lambda i: (0, 0)),
            pl.BlockSpec((1, D), lambda i: (0, 0)),
        ],
        out_specs=pl.BlockSpec((BF, D), lambda i: (i, 0)),
        out_shape=jax.ShapeDtypeStruct((E, D), jnp.float32),
    )(eij, edge_in_attr, res_e_w.T, stats, bn_e_g[None, :], bn_e_b[None, :])

    return (x_final, e_final)


# speedup vs baseline: 3.4732x; 1.0066x over previous
import functools

import jax
import jax.numpy as jnp
from jax import lax
from jax.experimental import pallas as pl
from jax.experimental.pallas import tpu as pltpu
from jax.experimental.pallas import tpu_sc as plsc

N = 10000
E = 320000
D = 128
DE = 16
EPS = 1e-5

NC = 2
NS = 16
NW = NC * NS
EPW = E // NW
K = 40
NCHUNK = EPW // K
ROWS_PT = 640
BN_BLK = 2000
BE = 4000
BF = 4000

_HIGH = lax.Precision.HIGHEST
_MED = lax.Precision.DEFAULT
_ILV = plsc.PackFormat.INTERLEAVED


def _pack_tc(lo_f32, hi_f32):
    lo = lax.bitcast_convert_type(lo_f32.astype(jnp.bfloat16),
                                  jnp.uint16).astype(jnp.uint32)
    hi = lax.bitcast_convert_type(hi_f32.astype(jnp.bfloat16),
                                  jnp.uint16).astype(jnp.uint32)
    return lax.bitcast_convert_type(lo | (hi << 16), jnp.float32)


def _unpack_tc(w_f32):
    u = lax.bitcast_convert_type(w_f32, jnp.uint32)
    lo = lax.bitcast_convert_type((u & 0xFFFF).astype(jnp.uint16),
                                  jnp.bfloat16).astype(jnp.float32)
    hi = lax.bitcast_convert_type((u >> 16).astype(jnp.uint16),
                                  jnp.bfloat16).astype(jnp.float32)
    return lo, hi


def _prep_body(x_ref, awt, ab, bwt, bb, dwt, db, ewt, eb_, ax_out, dtab_out,
               ebtab_out):
    x = x_ref[...]
    ax_out[...] = jnp.dot(x, awt[...], precision=_HIGH) + ab[...]
    dtab_out[...] = jnp.dot(x, dwt[...], precision=_HIGH) + db[...]
    ex = jnp.dot(x, ewt[...], precision=_HIGH) + eb_[...]
    bx = jnp.dot(x, bwt[...], precision=_HIGH) + bb[...]
    ebtab_out[...] = _pack_tc(ex, bx)


def _ce_body(attr_ref, cwt_ref, cb_ref, x_ref, awt, ab, bwt, bb, dwt, db,
             ewt, eb_, ce_out, ax_out, dtab_out, ebtab_out):
    cv = jnp.dot(attr_ref[...], cwt_ref[...], precision=_MED) + cb_ref[...]
    ce_out[...] = _pack_tc(cv[:, : D // 2], cv[:, D // 2:])

    @pl.when(pl.program_id(0) < N // BN_BLK)
    def _():
        _prep_body(x_ref, awt, ab, bwt, bb, dwt, db, ewt, eb_, ax_out,
                   dtab_out, ebtab_out)


def _sc_edge_body(row_hbm, col_hbm, dtab_hbm, ebtab_hbm, ce_hbm,
                  eij_hbm, aggr_hbm, stats_hbm,
                  row0, col0, gd0, geb0, eij0,
                  row1, col1, gd1, geb1, eij1,
                  ce_v, msg_v, st_v, aggr_sp, sem0, sem1, sem_ce,
                  sem_eij0, sem_eij1):
    c = lax.axis_index("c")
    s = lax.axis_index("s")
    wid = c * NS + s
    rows = (row0, row1)
    cols = (col0, col1)
    gds = (gd0, gd1)
    gebs = (geb0, geb1)
    eijs = (eij0, eij1)
    sems = (sem0, sem1)
    sems_eij = (sem_eij0, sem_eij1)

    def _zero_row(r, _):
        for j in range(D // 16):
            msg_v[r, pl.ds(j * 16, 16)] = jnp.zeros((16,), jnp.float32)
        return 0

    lax.fori_loop(0, K, _zero_row, 0)
    base_row = s * ROWS_PT
    nblk = jnp.where(s == NS - 1, (N - (NS - 1) * ROWS_PT) // K,
                     ROWS_PT // K)

    def _zero_blk(t, _):
        pltpu.sync_copy(msg_v, aggr_sp.at[pl.ds(base_row + t * K, K)])
        return 0

    lax.fori_loop(0, nblk, _zero_blk, 0)
    plsc.subcore_barrier()

    e0 = wid * EPW

    def _issue(b, ci):
        base = e0 + ci * K
        pltpu.sync_copy(row_hbm.at[pl.ds(base, K)], rows[b])
        pltpu.sync_copy(col_hbm.at[pl.ds(base, K)], cols[b])
        pltpu.async_copy(dtab_hbm.at[rows[b]], gds[b], sems[b])
        pltpu.async_copy(ebtab_hbm.at[cols[b]], gebs[b], sems[b])

    def _issue_ce(ci):
        base = e0 + ci * K
        pltpu.async_copy(ce_hbm.at[pl.ds(base, K)], ce_v, sem_ce)

    def _wait(b):
        pltpu.make_async_copy(dtab_hbm.at[rows[b]], gds[b], sems[b]).wait()
        pltpu.make_async_copy(ebtab_hbm.at[cols[b]], gebs[b],
                              sems[b]).wait()
        pltpu.make_async_copy(ce_hbm.at[pl.ds(0, K)], ce_v, sem_ce).wait()

    def _wait_eij(b):
        pltpu.make_async_copy(eijs[b], eij_hbm.at[pl.ds(0, K)],
                              sems_eij[b]).wait()

    def _compute(b, carry):
        gd_v, geb_v, eij_v = gds[b], gebs[b], eijs[b]

        def _row(r, rc):
            vs = list(rc)
            for j in range(D // 32):
                lo16 = pl.ds(j * 16, 16)
                hi16 = pl.ds(64 + j * 16, 16)
                c_lo, c_hi = plsc.unpack(
                    plsc.bitcast(ce_v[r, lo16], jnp.bfloat16), format=_ILV)
                e1, b1 = plsc.unpack(
                    plsc.bitcast(geb_v[r, lo16], jnp.bfloat16), format=_ILV)
                e2, b2 = plsc.unpack(
                    plsc.bitcast(geb_v[r, hi16], jnp.bfloat16), format=_ILV)
                elo = gd_v[r, lo16] + e1 + c_lo
                ehi = gd_v[r, hi16] + e2 + c_hi
                eij_v[r, lo16] = plsc.bitcast(
                    plsc.pack(elo, ehi, format=_ILV), jnp.float32)
                siglo = 1.0 / (1.0 + jnp.exp(-elo))
                sighi = 1.0 / (1.0 + jnp.exp(-ehi))
                msg_v[r, lo16] = siglo * b1
                msg_v[r, hi16] = sighi * b2
                vs[j] = vs[j] + elo
                vs[4 + j] = vs[4 + j] + ehi
                vs[8 + j] = vs[8 + j] + elo * elo
                vs[12 + j] = vs[12 + j] + ehi * ehi
            return tuple(vs)

        return lax.fori_loop(0, K, _row, carry)

    _issue(0, 0)
    _issue_ce(0)

    def _step(step, carry):
        for b in range(2):
            ci = 2 * step + b

            @pl.when(ci + 1 < NCHUNK)
            def _():
                _issue(1 - b, ci + 1)

            _wait(b)

            @pl.when(ci >= 2)
            def _():
                _wait_eij(b)

            carry = _compute(b, carry)

            @pl.when(ci + 1 < NCHUNK)
            def _():
                _issue_ce(ci + 1)

            base = e0 + ci * K
            pltpu.async_copy(eijs[b], eij_hbm.at[pl.ds(base, K)],
                             sems_eij[b])
            pltpu.sync_copy(msg_v, aggr_sp.at[rows[b]], add=True)
        return carry

    zeros = tuple(jnp.zeros((16,), jnp.float32) for _ in range(16))
    sums = lax.fori_loop(0, NCHUNK // 2, _step, zeros)
    _wait_eij(0)
    _wait_eij(1)

    for j in range(D // 32):
        st_v[0, pl.ds(j * 16, 16)] = sums[j]
        st_v[0, pl.ds(64 + j * 16, 16)] = sums[4 + j]
        st_v[1, pl.ds(j * 16, 16)] = sums[8 + j]
        st_v[1, pl.ds(64 + j * 16, 16)] = sums[12 + j]
    pltpu.sync_copy(st_v, stats_hbm.at[wid])

    plsc.subcore_barrier()

    def _out_blk(t, _):
        pltpu.sync_copy(aggr_sp.at[pl.ds(base_row + t * K, K)],
                        aggr_hbm.at[c, pl.ds(base_row + t * K, K)])
        return 0

    lax.fori_loop(0, nblk, _out_blk, 0)


def _node_body(x_ref, ax_ref, agg_ref, g_ref, b_ref, out_ref):
    t = ax_ref[...] + agg_ref[0] + agg_ref[1]
    m = jnp.mean(t, axis=0, keepdims=True)
    d = t - m
    v = jnp.mean(d * d, axis=0, keepdims=True)
    y = d * lax.rsqrt(v + EPS) * g_ref[...] + b_ref[...]
    out_ref[...] = x_ref[...] + jnp.maximum(y, 0.0)


def _efin_body(eij_ref, attr_ref, rwt_ref, stats_ref, g_ref, b_ref,
               x_ref, ax_ref, agg_ref, gx_ref, bx_ref, out_ref, x_out):
    @pl.when(pl.program_id(0) == 0)
    def _():
        _node_body(x_ref, ax_ref, agg_ref, gx_ref, bx_ref, x_out)

    ssum = jnp.sum(stats_ref[:, 0, :], axis=0, keepdims=True)
    ssq = jnp.sum(stats_ref[:, 1, :], axis=0, keepdims=True)
    m = ssum * (1.0 / E)
    v = ssq * (1.0 / E) - m * m
    scale = lax.rsqrt(v + EPS) * g_ref[...]
    elo, ehi = _unpack_tc(eij_ref[...])
    res = jnp.dot(attr_ref[...], rwt_ref[...], precision=_MED)
    h = D // 2
    ylo = (elo - m[:, :h]) * scale[:, :h] + b_ref[:, :h]
    yhi = (ehi - m[:, h:]) * scale[:, h:] + b_ref[:, h:]
    out_ref[:, :h] = res[:, :h] + jnp.maximum(ylo, 0.0)
    out_ref[:, h:] = res[:, h:] + jnp.maximum(yhi, 0.0)


def kernel(x_in_node, edge_idx, edge_in_attr, A_w, A_b, B_w, B_b, C_w, C_b,
           D_w, D_b, E_w, E_b, bn_x_g, bn_x_b, bn_e_g, bn_e_b, res_e_w):
    row = jnp.asarray(edge_idx[0], jnp.int32)
    col = jnp.asarray(edge_idx[1], jnp.int32)

    wspec = pl.BlockSpec((D, D), lambda i: (0, 0))
    bspec = pl.BlockSpec((1, D), lambda i: (0, 0))
    nblocks = N // BN_BLK
    nspec = pl.BlockSpec((BN_BLK, D),
                         lambda i: (jnp.minimum(i, nblocks - 1), 0))
    ce, ax, dtab, ebtab = pl.pallas_call(
        _ce_body,
        grid=(E // BE,),
        in_specs=[
            pl.BlockSpec((BE, DE), lambda i: (i, 0)),
            pl.BlockSpec((DE, D), lambda i: (0, 0)),
            pl.BlockSpec((1, D), lambda i: (0, 0)),
            nspec, wspec, bspec, wspec, bspec, wspec, bspec, wspec, bspec,
        ],
        out_specs=(pl.BlockSpec((BE, D // 2), lambda i: (i, 0)),
                   nspec, nspec, nspec),
        out_shape=(
            jax.ShapeDtypeStruct((E, D // 2), jnp.float32),
            jax.ShapeDtypeStruct((N, D), jnp.float32),
            jax.ShapeDtypeStruct((N, D), jnp.float32),
            jax.ShapeDtypeStruct((N, D), jnp.float32),
        ),
    )(edge_in_attr, C_w.T, C_b[None, :], x_in_node, A_w.T, A_b[None, :],
      B_w.T, B_b[None, :], D_w.T, D_b[None, :], E_w.T, E_b[None, :])

    sc_edge = functools.partial(
        pl.kernel,
        out_type=(
            jax.ShapeDtypeStruct((E, D // 2), jnp.float32),
            jax.ShapeDtypeStruct((NC, N, D), jnp.float32),
            jax.ShapeDtypeStruct((NW, 2, D), jnp.float32),
        ),
        mesh=plsc.VectorSubcoreMesh(core_axis_name="c", subcore_axis_name="s"),
        compiler_params=pltpu.CompilerParams(needs_layout_passes=False),
        scratch_types=[
            pltpu.VMEM((K,), jnp.int32),
            pltpu.VMEM((K,), jnp.int32),
            pltpu.VMEM((K, D), jnp.float32),
            pltpu.VMEM((K, D), jnp.float32),
            pltpu.VMEM((K, D // 2), jnp.float32),
            pltpu.VMEM((K,), jnp.int32),
            pltpu.VMEM((K,), jnp.int32),
            pltpu.VMEM((K, D), jnp.float32),
            pltpu.VMEM((K, D), jnp.float32),
            pltpu.VMEM((K, D // 2), jnp.float32),
            pltpu.VMEM((K, D // 2), jnp.float32),
            pltpu.VMEM((K, D), jnp.float32),
            pltpu.VMEM((2, D), jnp.float32),
            pltpu.VMEM_SHARED((N, D), jnp.float32),
            pltpu.SemaphoreType.DMA,
            pltpu.SemaphoreType.DMA,
            pltpu.SemaphoreType.DMA,
            pltpu.SemaphoreType.DMA,
            pltpu.SemaphoreType.DMA,
        ],
    )(_sc_edge_body)
    eij, aggr, stats = sc_edge(row, col, dtab, ebtab, ce)

    fullnode = pl.BlockSpec((N, D), lambda i: (0, 0))
    e_final, x_final = pl.pallas_call(
        _efin_body,
        grid=(E // BF,),
        in_specs=[
            pl.BlockSpec((BF, D // 2), lambda i: (i, 0)),
            pl.BlockSpec((BF, DE), lambda i: (i, 0)),
            pl.BlockSpec((DE, D), lambda i: (0, 0)),
            pl.BlockSpec((NW, 2, D), lambda i: (0, 0, 0)),
            pl.BlockSpec((1, D), lambda i: (0, 0)),
            pl.BlockSpec((1, D), lambda i: (0, 0)),
            fullnode,
            fullnode,
            pl.BlockSpec((NC, N, D), lambda i: (0, 0, 0)),
            pl.BlockSpec((1, D), lambda i: (0, 0)),
            pl.BlockSpec((1, D), lambda i: (0, 0)),
        ],
        out_specs=(pl.BlockSpec((BF, D), lambda i: (i, 0)), fullnode),
        out_shape=(
            jax.ShapeDtypeStruct((E, D), jnp.float32),
            jax.ShapeDtypeStruct((N, D), jnp.float32),
        ),
    )(eij, edge_in_attr, res_e_w.T, stats, bn_e_g[None, :], bn_e_b[None, :],
      x_in_node, ax, aggr, bn_x_g[None, :], bn_x_b[None, :])

    return (x_final, e_final)
